# baseline (speedup 1.0000x reference)
import functools

import jax
import jax.numpy as jnp
from jax import lax
from jax.experimental import pallas as pl
from jax.experimental.pallas import tpu as pltpu
from jax.experimental.pallas import tpu_sc as plsc

D_MODEL = 64
SCALE = 8.0
NUM_WORKERS = 32
BLK = 128


def _emb_body(xt_hbm, table_hbm, out_hbm, idx_v, rows_v, trans_v, gsem, *, seq_len):
    wid = lax.axis_index("s") * 2 + lax.axis_index("c")
    b0 = wid * BLK

    pltpu.sync_copy(xt_hbm.at[:, pl.ds(b0, BLK)], idx_v)

    lane = lax.iota(jnp.int32, 16)

    @pl.loop(0, seq_len)
    def _pos(l):
        pltpu.async_copy(table_hbm.at[idx_v.at[l]], rows_v, gsem).wait()

        for j in range(D_MODEL):
            for c in range(BLK // 16):
                v = plsc.load_gather(
                    rows_v, [lane + (16 * c), jnp.full((16,), j, jnp.int32)]
                )
                trans_v[j, pl.ds(16 * c, 16)] = v * SCALE

        pltpu.sync_copy(trans_v, out_hbm.at[l, :, pl.ds(b0, BLK)])


def kernel(x, table):
    B, L = x.shape
    V, D = table.shape
    assert D == D_MODEL and B == NUM_WORKERS * BLK

    xt = x.T
    table_p = jnp.pad(table, ((0, 0), (0, 128 - D)))

    mesh = plsc.VectorSubcoreMesh(core_axis_name="c", subcore_axis_name="s")

    emb = functools.partial(
        pl.kernel,
        out_type=jax.ShapeDtypeStruct((L, D_MODEL, B), jnp.float32),
        mesh=mesh,
        compiler_params=pltpu.CompilerParams(
            use_tc_tiling_on_sc=True, needs_layout_passes=False
        ),
        scratch_types=[
            pltpu.VMEM((L, BLK), jnp.int32),
            pltpu.VMEM((BLK, 128), jnp.float32),
            pltpu.VMEM((D_MODEL, BLK), jnp.float32),
            pltpu.SemaphoreType.DMA,
        ],
    )(functools.partial(_emb_body, seq_len=L))

    out_p = emb(xt, table_p)
    return out_p.transpose(2, 0, 1)


# speedup vs baseline: 1.8842x; 1.8842x over previous
import functools

import jax
import jax.numpy as jnp
from jax import lax
from jax.experimental import pallas as pl
from jax.experimental.pallas import tpu as pltpu
from jax.experimental.pallas import tpu_sc as plsc

D_MODEL = 64
SCALE = 8.0
NUM_WORKERS = 32
BLK = 128


def _emb_body(
    xt_hbm, table_hbm, out_hbm,
    idx_v, rows_a, rows_b, trans_a, trans_b, gsem, osem,
    *, seq_len,
):
    wid = lax.axis_index("s") * 2 + lax.axis_index("c")
    b0 = wid * BLK
    rows = (rows_a, rows_b)
    trans = (trans_a, trans_b)

    pltpu.sync_copy(xt_hbm.at[:, pl.ds(b0, BLK)], idx_v)

    lane = lax.iota(jnp.int32, 16)
    zeros = jnp.zeros((16,), jnp.int32)
    bvecs = [lane + 16 * c for c in range(BLK // 16)]

    def out_dst(l):
        return out_hbm.at[l, :, pl.ds(b0, BLK)]

    def fire_gather(l, ph):
        pltpu.async_copy(table_hbm.at[idx_v.at[l]], rows[ph], gsem)

    fire_gather(0, 0)
    fire_gather(1, 1)

    @pl.loop(0, seq_len // 2)
    def _pair(p):
        for ph in range(2):
            l = 2 * p + ph

            @pl.when(l >= 2)
            def _():
                pltpu.make_async_copy(trans[ph], out_dst(l - 2), osem).wait()

            pltpu.make_async_copy(table_hbm.at[idx_v.at[l]], rows[ph], gsem).wait()

            @plsc.parallel_loop(0, D_MODEL, unroll=4)
            def _t(j):
                jvec = zeros + j
                for c in range(BLK // 16):
                    v = plsc.load_gather(rows[ph], [bvecs[c], jvec])
                    trans[ph][j, pl.ds(16 * c, 16)] = v * SCALE

            @pl.when(l + 2 < seq_len)
            def _():
                fire_gather(l + 2, ph)

            pltpu.async_copy(trans[ph], out_dst(l), osem)

    for ph in range(2):
        l = seq_len - 2 + ph
        pltpu.make_async_copy(trans[ph], out_dst(l), osem).wait()


def kernel(x, table):
    B, L = x.shape
    V, D = table.shape
    assert D == D_MODEL and B == NUM_WORKERS * BLK and L % 2 == 0

    xt = x.T
    table_p = jnp.pad(table, ((0, 0), (0, 128 - D)))

    mesh = plsc.VectorSubcoreMesh(core_axis_name="c", subcore_axis_name="s")

    emb = functools.partial(
        pl.kernel,
        out_type=jax.ShapeDtypeStruct((L, D_MODEL, B), jnp.float32),
        mesh=mesh,
        compiler_params=pltpu.CompilerParams(
            use_tc_tiling_on_sc=True, needs_layout_passes=False
        ),
        scratch_types=[
            pltpu.VMEM((L, BLK), jnp.int32),
            pltpu.VMEM((BLK, 128), jnp.float32),
            pltpu.VMEM((BLK, 128), jnp.float32),
            pltpu.VMEM((D_MODEL, BLK), jnp.float32),
            pltpu.VMEM((D_MODEL, BLK), jnp.float32),
            pltpu.SemaphoreType.DMA,
            pltpu.SemaphoreType.DMA,
        ],
    )(functools.partial(_emb_body, seq_len=L))

    out_p = emb(xt, table_p)
    return out_p.transpose(2, 0, 1)


# speedup vs baseline: 1.8872x; 1.0016x over previous
import functools

import jax
import jax.numpy as jnp
from jax import lax
from jax.experimental import pallas as pl
from jax.experimental.pallas import tpu as pltpu
from jax.experimental.pallas import tpu_sc as plsc

D_MODEL = 64
SCALE = 8.0
NUM_WORKERS = 32
BLK = 128


NBUF = 4


def _emb_body(
    xt_hbm, table_hbm, out_hbm,
    idx_v, rows_a, rows_b, rows_c, rows_d, trans_a, trans_b, gsem, osem,
    *, seq_len,
):
    wid = lax.axis_index("s") * 2 + lax.axis_index("c")
    b0 = wid * BLK
    rows = (rows_a, rows_b, rows_c, rows_d)
    trans = (trans_a, trans_b)

    pltpu.sync_copy(xt_hbm.at[:, pl.ds(b0, BLK)], idx_v)

    lane = lax.iota(jnp.int32, 16)
    zeros = jnp.zeros((16,), jnp.int32)
    bvecs = [lane + 16 * c for c in range(BLK // 16)]

    def out_dst(l):
        return out_hbm.at[l, :, pl.ds(b0, BLK)]

    def fire_gather(l, ph):
        pltpu.async_copy(table_hbm.at[idx_v.at[l]], rows[ph], gsem)

    for ph in range(NBUF):
        fire_gather(ph, ph)

    @pl.loop(0, seq_len // NBUF)
    def _grp(p):
        for ph in range(NBUF):
            l = NBUF * p + ph
            tb = ph % 2

            @pl.when(l >= 2)
            def _():
                pltpu.make_async_copy(trans[tb], out_dst(l - 2), osem).wait()

            pltpu.make_async_copy(table_hbm.at[idx_v.at[l]], rows[ph], gsem).wait()

            @plsc.parallel_loop(0, D_MODEL, unroll=8)
            def _t(j):
                jvec = zeros + j
                for c in range(BLK // 16):
                    v = plsc.load_gather(rows[ph], [bvecs[c], jvec])
                    trans[tb][j, pl.ds(16 * c, 16)] = v * SCALE

            @pl.when(l + NBUF < seq_len)
            def _():
                fire_gather(l + NBUF, ph)

            pltpu.async_copy(trans[tb], out_dst(l), osem)

    for l in (seq_len - 2, seq_len - 1):
        pltpu.make_async_copy(trans[l % 2], out_dst(l), osem).wait()


def kernel(x, table):
    B, L = x.shape
    V, D = table.shape
    assert D == D_MODEL and B == NUM_WORKERS * BLK and L % 2 == 0

    xt = x.T
    table_p = jnp.pad(table, ((0, 0), (0, 128 - D)))

    mesh = plsc.VectorSubcoreMesh(core_axis_name="c", subcore_axis_name="s")

    emb = functools.partial(
        pl.kernel,
        out_type=jax.ShapeDtypeStruct((L, D_MODEL, B), jnp.float32),
        mesh=mesh,
        compiler_params=pltpu.CompilerParams(
            use_tc_tiling_on_sc=True, needs_layout_passes=False
        ),
        scratch_types=[
            pltpu.VMEM((L, BLK), jnp.int32),
            pltpu.VMEM((BLK, 128), jnp.float32),
            pltpu.VMEM((BLK, 128), jnp.float32),
            pltpu.VMEM((BLK, 128), jnp.float32),
            pltpu.VMEM((BLK, 128), jnp.float32),
            pltpu.VMEM((D_MODEL, BLK), jnp.float32),
            pltpu.VMEM((D_MODEL, BLK), jnp.float32),
            pltpu.SemaphoreType.DMA,
            pltpu.SemaphoreType.DMA,
        ],
    )(functools.partial(_emb_body, seq_len=L))

    out_p = emb(xt, table_p)
    return out_p.transpose(2, 0, 1)


# speedup vs baseline: 2.3762x; 1.2591x over previous
import functools

import jax
import jax.numpy as jnp
from jax import lax
from jax.experimental import pallas as pl
from jax.experimental.pallas import tpu as pltpu
from jax.experimental.pallas import tpu_sc as plsc

D_MODEL = 64
SCALE = 8.0
NUM_WORKERS = 32
BLK = 128


NBUF = 4


def _emb_body(
    xt_hbm, table_hbm, out_hbm,
    idx_v, rows_a, rows_b, rows_c, rows_d, trans_a, trans_b, gsem, osem,
    *, seq_len,
):
    wid = lax.axis_index("s") * 2 + lax.axis_index("c")
    b0 = wid * BLK
    rows = (rows_a, rows_b, rows_c, rows_d)
    trans = (trans_a, trans_b)

    pltpu.sync_copy(xt_hbm.at[:, pl.ds(b0, BLK)], idx_v)

    lane = lax.iota(jnp.int32, 16)
    bvecs = [lane + 16 * c for c in range(BLK // 16)]
    rotvecs = [(lane + d) & 15 for d in range(16)]

    def out_dst(l):
        return out_hbm.at[l, :, pl.ds(b0, BLK)]

    def fire_gather(l, ph):
        pltpu.async_copy(table_hbm.at[idx_v.at[l]], rows[ph], gsem)

    for ph in range(NBUF):
        fire_gather(ph, ph)

    @pl.loop(0, seq_len // NBUF)
    def _grp(p):
        for ph in range(NBUF):
            l = NBUF * p + ph
            tb = ph % 2

            @pl.when(l >= 2)
            def _():
                pltpu.make_async_copy(trans[tb], out_dst(l - 2), osem).wait()

            pltpu.make_async_copy(table_hbm.at[idx_v.at[l]], rows[ph], gsem).wait()

            @plsc.parallel_loop(0, (BLK // 16) * (D_MODEL // 16), unroll=2)
            def _t(i):
                c = i >> 2
                j16 = (i & 3) << 4
                bvec = lane + (c << 4)
                for d in range(16):
                    jvec = j16 + rotvecs[d]
                    v = plsc.load_gather(rows[ph], [bvec, jvec])
                    plsc.store_scatter(trans[tb], [jvec, bvec], v * SCALE)

            @pl.when(l + NBUF < seq_len)
            def _():
                fire_gather(l + NBUF, ph)

            pltpu.async_copy(trans[tb], out_dst(l), osem)

    for l in (seq_len - 2, seq_len - 1):
        pltpu.make_async_copy(trans[l % 2], out_dst(l), osem).wait()


def kernel(x, table):
    B, L = x.shape
    V, D = table.shape
    assert D == D_MODEL and B == NUM_WORKERS * BLK and L % 2 == 0

    xt = x.T
    table_p = jnp.pad(table, ((0, 0), (0, 128 - D)))

    mesh = plsc.VectorSubcoreMesh(core_axis_name="c", subcore_axis_name="s")

    emb = functools.partial(
        pl.kernel,
        out_type=jax.ShapeDtypeStruct((L, D_MODEL, B), jnp.float32),
        mesh=mesh,
        compiler_params=pltpu.CompilerParams(
            use_tc_tiling_on_sc=True, needs_layout_passes=False
        ),
        scratch_types=[
            pltpu.VMEM((L, BLK), jnp.int32),
            pltpu.VMEM((BLK, 128), jnp.float32),
            pltpu.VMEM((BLK, 128), jnp.float32),
            pltpu.VMEM((BLK, 128), jnp.float32),
            pltpu.VMEM((BLK, 128), jnp.float32),
            pltpu.VMEM((D_MODEL, BLK), jnp.float32),
            pltpu.VMEM((D_MODEL, BLK), jnp.float32),
            pltpu.SemaphoreType.DMA,
            pltpu.SemaphoreType.DMA,
        ],
    )(functools.partial(_emb_body, seq_len=L))

    out_p = emb(xt, table_p)
    return out_p.transpose(2, 0, 1)


# speedup vs baseline: 3.8719x; 1.6294x over previous
import functools

import jax
import jax.numpy as jnp
from jax import lax
from jax.experimental import pallas as pl
from jax.experimental.pallas import tpu as pltpu
from jax.experimental.pallas import tpu_sc as plsc

D_MODEL = 64
SCALE = 8.0
NUM_WORKERS = 32
BLK = 128
NBUF = 4
PREP_T = 246


def _wid():
    return lax.axis_index("s") * 2 + lax.axis_index("c")


def _diag_vecs():
    lane = lax.iota(jnp.int32, 16)
    bvecs = [lane + 16 * c for c in range(8)]
    rotvecs = [(lane + d) & 15 for d in range(16)]
    return bvecs, rotvecs


def _prep_body(tt_hbm, out_hbm, vin_a, vin_b, vout_a, vout_b, vtail, isem, osem, *, vocab):
    wid = _wid()
    n_full = vocab // 128
    vin = (vin_a, vin_b)
    vout = (vout_a, vout_b)
    bvecs, rotvecs = _diag_vecs()

    def i0_of(t):
        blk = wid + NUM_WORKERS * t
        return pl.multiple_of(blk * 128, 128)

    def valid(t):
        return (wid + NUM_WORKERS * t) < n_full

    def fire_in(t, ph):
        pltpu.async_copy(tt_hbm.at[:, pl.ds(i0_of(t), 128)], vin[ph], isem)

    for t in range(2):
        @pl.when(valid(t))
        def _():
            fire_in(t, t)

    @pl.loop(0, PREP_T // 2)
    def _pair(p):
        for ph in range(2):
            t = 2 * p + ph

            @pl.when(valid(t))
            def _():
                @pl.when(t >= 2)
                def _():
                    pltpu.async_copy(
                        vout[ph], out_hbm.at[pl.ds(i0_of(t - 2), 128)], osem
                    ).wait()

                pltpu.make_async_copy(
                    tt_hbm.at[:, pl.ds(i0_of(t), 128)], vin[ph], isem
                ).wait()

                @plsc.parallel_loop(0, 32, unroll=2)
                def _t(i):
                    c2 = i >> 2
                    j16 = (i & 3) << 4
                    ivec = bvecs[0] + (c2 << 4)
                    for d in range(16):
                        jvec = j16 + rotvecs[d]
                        v = plsc.load_gather(vin[ph], [jvec, ivec])
                        plsc.store_scatter(vout[ph], [ivec, jvec], v * SCALE)

                @pl.when(valid(t + 2))
                def _():
                    fire_in(t + 2, ph)

                pltpu.async_copy(vout[ph], out_hbm.at[pl.ds(i0_of(t), 128)], osem)

    for ph in range(2):
        t_last = PREP_T - 2 + ph

        @pl.when(valid(t_last))
        def _():
            pltpu.make_async_copy(
                vout[ph], out_hbm.at[pl.ds(i0_of(t_last), 128)], osem
            ).wait()

    @pl.when(wid == 0)
    def _tail():
        tail0 = (vocab // 128) * 128
        n_tail = vocab - tail0
        pltpu.sync_copy(tt_hbm.at[:, pl.ds(tail0, n_tail)], vtail)

        @pl.loop(0, (n_tail // 16) * 4)
        def _tt(i):
            c2 = i >> 2
            j16 = (i & 3) << 4
            ivec = bvecs[0] + (c2 << 4)
            for d in range(16):
                jvec = j16 + rotvecs[d]
                v = plsc.load_gather(vtail, [jvec, ivec])
                plsc.store_scatter(vout[0], [ivec, jvec], v * SCALE)
        pltpu.sync_copy(
            vout[0].at[pl.ds(0, n_tail)],
            out_hbm.at[pl.ds(tail0, n_tail)],
        )


def _emb_body(
    xt_hbm, table_hbm, out_hbm,
    idx_v, rows_a, rows_b, rows_c, rows_d, trans_a, trans_b, gsem, osem,
    *, seq_len,
):
    wid = _wid()
    b0 = wid * BLK
    rows = (rows_a, rows_b, rows_c, rows_d)
    trans = (trans_a, trans_b)

    pltpu.sync_copy(xt_hbm.at[:, pl.ds(b0, BLK)], idx_v)

    bvecs, rotvecs = _diag_vecs()

    def out_dst(l):
        return out_hbm.at[l, :, pl.ds(b0, BLK)]

    def fire_gather(l, ph):
        pltpu.async_copy(table_hbm.at[idx_v.at[l]], rows[ph], gsem)

    for ph in range(NBUF):
        fire_gather(ph, ph)

    @pl.loop(0, seq_len // NBUF)
    def _grp(p):
        for ph in range(NBUF):
            l = NBUF * p + ph
            tb = ph % 2

            @pl.when(l >= 2)
            def _():
                pltpu.make_async_copy(trans[tb], out_dst(l - 2), osem).wait()

            pltpu.make_async_copy(table_hbm.at[idx_v.at[l]], rows[ph], gsem).wait()

            @plsc.parallel_loop(0, 32, unroll=2)
            def _t(i):
                c = i >> 2
                j16 = (i & 3) << 4
                bvec = bvecs[0] + (c << 4)
                for d in range(16):
                    jvec = j16 + rotvecs[d]
                    v = plsc.load_gather(rows[ph], [bvec, jvec])
                    plsc.store_scatter(trans[tb], [jvec, bvec], v)

            @pl.when(l + NBUF < seq_len)
            def _():
                fire_gather(l + NBUF, ph)

            pltpu.async_copy(trans[tb], out_dst(l), osem)

    for l in (seq_len - 2, seq_len - 1):
        pltpu.make_async_copy(trans[l % 2], out_dst(l), osem).wait()


def kernel(x, table):
    B, L = x.shape
    V, D = table.shape
    assert D == D_MODEL and B == NUM_WORKERS * BLK and L % NBUF == 0

    mesh = plsc.VectorSubcoreMesh(core_axis_name="c", subcore_axis_name="s")
    params = pltpu.CompilerParams(use_tc_tiling_on_sc=True, needs_layout_passes=False)

    prep = functools.partial(
        pl.kernel,
        out_type=jax.ShapeDtypeStruct((V, 128), jnp.float32),
        mesh=mesh,
        compiler_params=params,
        scratch_types=[
            pltpu.VMEM((D_MODEL, 128), jnp.float32),
            pltpu.VMEM((D_MODEL, 128), jnp.float32),
            pltpu.VMEM((128, 128), jnp.float32),
            pltpu.VMEM((128, 128), jnp.float32),
            pltpu.VMEM((D_MODEL, 64), jnp.float32),
            pltpu.SemaphoreType.DMA,
            pltpu.SemaphoreType.DMA,
        ],
    )(functools.partial(_prep_body, vocab=V))

    emb = functools.partial(
        pl.kernel,
        out_type=jax.ShapeDtypeStruct((L, D_MODEL, B), jnp.float32),
        mesh=mesh,
        compiler_params=params,
        scratch_types=[
            pltpu.VMEM((L, BLK), jnp.int32),
            pltpu.VMEM((BLK, 128), jnp.float32),
            pltpu.VMEM((BLK, 128), jnp.float32),
            pltpu.VMEM((BLK, 128), jnp.float32),
            pltpu.VMEM((BLK, 128), jnp.float32),
            pltpu.VMEM((D_MODEL, BLK), jnp.float32),
            pltpu.VMEM((D_MODEL, BLK), jnp.float32),
            pltpu.SemaphoreType.DMA,
            pltpu.SemaphoreType.DMA,
        ],
    )(functools.partial(_emb_body, seq_len=L))

    table_p = prep(table.T)
    out_p = emb(x.T, table_p)
    return out_p.transpose(2, 0, 1)


# speedup vs baseline: 4.0907x; 1.0565x over previous
import functools

import jax
import jax.numpy as jnp
from jax import lax
from jax.experimental import pallas as pl
from jax.experimental.pallas import tpu as pltpu
from jax.experimental.pallas import tpu_sc as plsc

D_MODEL = 64
SCALE = 8.0
NUM_WORKERS = 32
BLK = 128
NBUF = 4
PREP_T = 246


def _wid():
    return lax.axis_index("s") * 2 + lax.axis_index("c")


def _diag_vecs():
    lane = lax.iota(jnp.int32, 16)
    bvecs = [lane + 16 * c for c in range(8)]
    rotvecs = [(lane + d) & 15 for d in range(16)]
    return bvecs, rotvecs


def _prep_body(tt_hbm, out_hbm, vin_a, vin_b, vout_a, vout_b, vtail, isem_a, isem_b, osem_a, osem_b, *, vocab):
    wid = _wid()
    n_full = vocab // 128
    vin = (vin_a, vin_b)
    vout = (vout_a, vout_b)
    isem = (isem_a, isem_b)
    osem = (osem_a, osem_b)
    bvecs, rotvecs = _diag_vecs()

    def i0_of(t):
        blk = wid + NUM_WORKERS * t
        return pl.multiple_of(blk * 128, 128)

    def valid(t):
        return (wid + NUM_WORKERS * t) < n_full

    def fire_in(t, ph):
        pltpu.async_copy(tt_hbm.at[:, pl.ds(i0_of(t), 128)], vin[ph], isem[ph])

    for t in range(2):
        @pl.when(valid(t))
        def _():
            fire_in(t, t)

    @pl.loop(0, PREP_T // 2)
    def _pair(p):
        for ph in range(2):
            t = 2 * p + ph

            @pl.when(valid(t))
            def _():
                @pl.when(t >= 2)
                def _():
                    pltpu.make_async_copy(
                        vout[ph], out_hbm.at[pl.ds(i0_of(t - 2), 128)], osem[ph]
                    ).wait()

                pltpu.make_async_copy(
                    tt_hbm.at[:, pl.ds(i0_of(t), 128)], vin[ph], isem[ph]
                ).wait()

                @plsc.parallel_loop(0, 32, unroll=2)
                def _t(i):
                    c2 = i >> 2
                    j16 = (i & 3) << 4
                    ivec = bvecs[0] + (c2 << 4)
                    for d in range(16):
                        jvec = j16 + rotvecs[d]
                        v = plsc.load_gather(vin[ph], [jvec, ivec])
                        plsc.store_scatter(vout[ph], [ivec, jvec], v * SCALE)

                @pl.when(valid(t + 2))
                def _():
                    fire_in(t + 2, ph)

                pltpu.async_copy(vout[ph], out_hbm.at[pl.ds(i0_of(t), 128)], osem[ph])

    for ph in range(2):
        t_last = PREP_T - 2 + ph

        @pl.when(valid(t_last))
        def _():
            pltpu.make_async_copy(
                vout[ph], out_hbm.at[pl.ds(i0_of(t_last), 128)], osem[ph]
            ).wait()

    @pl.when(wid == 0)
    def _tail():
        tail0 = (vocab // 128) * 128
        n_tail = vocab - tail0
        pltpu.sync_copy(tt_hbm.at[:, pl.ds(tail0, n_tail)], vtail)

        @pl.loop(0, (n_tail // 16) * 4)
        def _tt(i):
            c2 = i >> 2
            j16 = (i & 3) << 4
            ivec = bvecs[0] + (c2 << 4)
            for d in range(16):
                jvec = j16 + rotvecs[d]
                v = plsc.load_gather(vtail, [jvec, ivec])
                plsc.store_scatter(vout[0], [ivec, jvec], v * SCALE)
        pltpu.sync_copy(
            vout[0].at[pl.ds(0, n_tail)],
            out_hbm.at[pl.ds(tail0, n_tail)],
        )


def _emb_body(
    xt_hbm, table_hbm, out_hbm,
    idx_v, rows_a, rows_b, rows_c, rows_d, trans_a, trans_b,
    gsem_a, gsem_b, gsem_c, gsem_d, osem_a, osem_b,
    *, seq_len,
):
    wid = _wid()
    b0 = wid * BLK
    rows = (rows_a, rows_b, rows_c, rows_d)
    trans = (trans_a, trans_b)
    gsem = (gsem_a, gsem_b, gsem_c, gsem_d)
    osem = (osem_a, osem_b)

    pltpu.sync_copy(xt_hbm.at[:, pl.ds(b0, BLK)], idx_v)

    bvecs, rotvecs = _diag_vecs()

    def out_dst(l):
        return out_hbm.at[l, :, pl.ds(b0, BLK)]

    def fire_gather(l, ph):
        pltpu.async_copy(table_hbm.at[idx_v.at[l]], rows[ph], gsem[ph])

    for ph in range(NBUF):
        fire_gather(ph, ph)

    @pl.loop(0, seq_len // NBUF)
    def _grp(p):
        for ph in range(NBUF):
            l = NBUF * p + ph
            tb = ph % 2

            @pl.when(l >= 2)
            def _():
                pltpu.make_async_copy(trans[tb], out_dst(l - 2), osem[tb]).wait()

            pltpu.make_async_copy(table_hbm.at[idx_v.at[l]], rows[ph], gsem[ph]).wait()

            @plsc.parallel_loop(0, 32, unroll=2)
            def _t(i):
                c = i >> 2
                j16 = (i & 3) << 4
                bvec = bvecs[0] + (c << 4)
                for d in range(16):
                    jvec = j16 + rotvecs[d]
                    v = plsc.load_gather(rows[ph], [bvec, jvec])
                    plsc.store_scatter(trans[tb], [jvec, bvec], v)

            @pl.when(l + NBUF < seq_len)
            def _():
                fire_gather(l + NBUF, ph)

            pltpu.async_copy(trans[tb], out_dst(l), osem[tb])

    for l in (seq_len - 2, seq_len - 1):
        pltpu.make_async_copy(trans[l % 2], out_dst(l), osem[l % 2]).wait()


def kernel(x, table):
    B, L = x.shape
    V, D = table.shape
    assert D == D_MODEL and B == NUM_WORKERS * BLK and L % NBUF == 0

    mesh = plsc.VectorSubcoreMesh(core_axis_name="c", subcore_axis_name="s")
    params = pltpu.CompilerParams(use_tc_tiling_on_sc=True, needs_layout_passes=False)

    prep = functools.partial(
        pl.kernel,
        out_type=jax.ShapeDtypeStruct((V, 128), jnp.float32),
        mesh=mesh,
        compiler_params=params,
        scratch_types=[
            pltpu.VMEM((D_MODEL, 128), jnp.float32),
            pltpu.VMEM((D_MODEL, 128), jnp.float32),
            pltpu.VMEM((128, 128), jnp.float32),
            pltpu.VMEM((128, 128), jnp.float32),
            pltpu.VMEM((D_MODEL, 64), jnp.float32),
            pltpu.SemaphoreType.DMA,
            pltpu.SemaphoreType.DMA,
            pltpu.SemaphoreType.DMA,
            pltpu.SemaphoreType.DMA,
        ],
    )(functools.partial(_prep_body, vocab=V))

    emb = functools.partial(
        pl.kernel,
        out_type=jax.ShapeDtypeStruct((L, D_MODEL, B), jnp.float32),
        mesh=mesh,
        compiler_params=params,
        scratch_types=[
            pltpu.VMEM((L, BLK), jnp.int32),
            pltpu.VMEM((BLK, 128), jnp.float32),
            pltpu.VMEM((BLK, 128), jnp.float32),
            pltpu.VMEM((BLK, 128), jnp.float32),
            pltpu.VMEM((BLK, 128), jnp.float32),
            pltpu.VMEM((D_MODEL, BLK), jnp.float32),
            pltpu.VMEM((D_MODEL, BLK), jnp.float32),
            pltpu.SemaphoreType.DMA,
            pltpu.SemaphoreType.DMA,
            pltpu.SemaphoreType.DMA,
            pltpu.SemaphoreType.DMA,
            pltpu.SemaphoreType.DMA,
            pltpu.SemaphoreType.DMA,
        ],
    )(functools.partial(_emb_body, seq_len=L))

    table_p = prep(table.T)
    out_p = emb(x.T, table_p)
    return out_p.transpose(2, 0, 1)
